# baseline (speedup 1.0000x reference)
import jax
import jax.numpy as jnp
from jax.experimental import pallas as pl
from jax.experimental.pallas import tpu as pltpu

_NBUF = 3


def kernel(inputs, embed_weight):
    B, T, D = inputs.shape
    Q = T // 4
    x2 = inputs.reshape(B * T, D)
    jobs = [(q * Q, Q, q * Q) for q in range(4)]
    for b in range(1, B - 1):
        jobs.append((b * T, T, 0))
    jobs += [((B - 1) * T + q * Q, Q, q * Q) for q in range(4)]
    NJ = len(jobs)

    def body(x_hbm, w_hbm, o_hbm, wvm, xb0, xb1, xb2, wsem, isem, osem):
        xbufs = (xb0, xb1, xb2)

        def start_in(j):
            r0, n, _ = jobs[j]
            cp = pltpu.make_async_copy(
                x_hbm.at[pl.ds(r0, n)], xbufs[j % _NBUF].at[pl.ds(0, n)],
                isem.at[j % _NBUF])
            cp.start()
            return cp

        def start_out(j):
            r0, n, _ = jobs[j]
            cp = pltpu.make_async_copy(
                xbufs[j % _NBUF].at[pl.ds(0, n)], o_hbm.at[pl.ds(r0, n)],
                osem.at[j % _NBUF])
            cp.start()
            return cp

        wcps = []
        for h in range(4):
            cp = pltpu.make_async_copy(
                w_hbm.at[pl.ds(h * Q, Q)], wvm.at[pl.ds(h * Q, Q)],
                wsem.at[h])
            cp.start()
            wcps.append(cp)
        incp = [None] * NJ
        outcp = [None] * NJ
        for j in range(_NBUF - 1):
            incp[j] = start_in(j)
        for j in range(NJ):
            r0, n, woff = jobs[j]
            if j < 4:
                wcps[j].wait()
            incp[j].wait()
            xbuf = xbufs[j % _NBUF]
            xbuf[pl.ds(0, n)] = xbuf[pl.ds(0, n)] + wvm[pl.ds(woff, n)]
            outcp[j] = start_out(j)
            nxt = j + _NBUF - 1
            if nxt < NJ:
                if nxt >= _NBUF:
                    outcp[nxt - _NBUF].wait()
                incp[nxt] = start_in(nxt)
        for j in range(max(0, NJ - _NBUF), NJ):
            outcp[j].wait()

    out2 = pl.pallas_call(
        body,
        grid=(1,),
        in_specs=[
            pl.BlockSpec(memory_space=pl.ANY),
            pl.BlockSpec(memory_space=pl.ANY),
        ],
        out_specs=pl.BlockSpec(memory_space=pl.ANY),
        out_shape=jax.ShapeDtypeStruct((B * T, D), inputs.dtype),
        scratch_shapes=[
            pltpu.VMEM((T, D), jnp.float32),
            pltpu.VMEM((T, D), jnp.float32),
            pltpu.VMEM((T, D), jnp.float32),
            pltpu.VMEM((T, D), jnp.float32),
            pltpu.SemaphoreType.DMA((4,)),
            pltpu.SemaphoreType.DMA((_NBUF,)),
            pltpu.SemaphoreType.DMA((_NBUF,)),
        ],
    )(x2, embed_weight)
    return out2.reshape(B, T, D)


# speedup vs baseline: 1.0916x; 1.0916x over previous
import jax
import jax.numpy as jnp
from jax.experimental import pallas as pl
from jax.experimental.pallas import tpu as pltpu

_NBUF = 4


def kernel(inputs, embed_weight):
    B, T, D = inputs.shape
    H = T // 2
    x2 = inputs.reshape(B * T, D)
    jobs = [(h * H, H, h * H) for h in range(2)]
    for b in range(1, B - 1):
        jobs.append((b * T, T, 0))
    jobs += [((B - 1) * T + h * H, H, h * H) for h in range(2)]
    NJ = len(jobs)

    def body(x_hbm, w_hbm, o_hbm, wvm, xb0, xb1, xb2, xb3, wsem, isem, osem):
        xbufs = (xb0, xb1, xb2, xb3)

        def start_in(j):
            r0, n, _ = jobs[j]
            cp = pltpu.make_async_copy(
                x_hbm.at[pl.ds(r0, n)], xbufs[j % _NBUF].at[pl.ds(0, n)],
                isem.at[j % _NBUF])
            cp.start()
            return cp

        def start_out(j):
            r0, n, _ = jobs[j]
            cp = pltpu.make_async_copy(
                xbufs[j % _NBUF].at[pl.ds(0, n)], o_hbm.at[pl.ds(r0, n)],
                osem.at[j % _NBUF])
            cp.start()
            return cp

        wcps = []
        for h in range(2):
            cp = pltpu.make_async_copy(
                w_hbm.at[pl.ds(h * H, H)], wvm.at[pl.ds(h * H, H)],
                wsem.at[h])
            cp.start()
            wcps.append(cp)
        incp = [None] * NJ
        outcp = [None] * NJ
        for j in range(_NBUF - 1):
            incp[j] = start_in(j)
        for j in range(NJ):
            r0, n, woff = jobs[j]
            if j < 2:
                wcps[j].wait()
            incp[j].wait()
            xbuf = xbufs[j % _NBUF]
            xbuf[pl.ds(0, n)] = xbuf[pl.ds(0, n)] + wvm[pl.ds(woff, n)]
            outcp[j] = start_out(j)
            nxt = j + _NBUF - 1
            if nxt < NJ:
                if nxt >= _NBUF:
                    outcp[nxt - _NBUF].wait()
                incp[nxt] = start_in(nxt)
        for j in range(max(0, NJ - _NBUF), NJ):
            outcp[j].wait()

    out2 = pl.pallas_call(
        body,
        grid=(1,),
        in_specs=[
            pl.BlockSpec(memory_space=pl.ANY),
            pl.BlockSpec(memory_space=pl.ANY),
        ],
        out_specs=pl.BlockSpec(memory_space=pl.ANY),
        out_shape=jax.ShapeDtypeStruct((B * T, D), inputs.dtype),
        scratch_shapes=[
            pltpu.VMEM((T, D), jnp.float32),
            pltpu.VMEM((T, D), jnp.float32),
            pltpu.VMEM((T, D), jnp.float32),
            pltpu.VMEM((T, D), jnp.float32),
            pltpu.VMEM((T, D), jnp.float32),
            pltpu.SemaphoreType.DMA((2,)),
            pltpu.SemaphoreType.DMA((_NBUF,)),
            pltpu.SemaphoreType.DMA((_NBUF,)),
        ],
    )(x2, embed_weight)
    return out2.reshape(B, T, D)


# speedup vs baseline: 1.0952x; 1.0033x over previous
import jax
import jax.numpy as jnp
from jax.experimental import pallas as pl
from jax.experimental.pallas import tpu as pltpu

_NBUF = 5


def kernel(inputs, embed_weight):
    B, T, D = inputs.shape
    H = T // 2
    x2 = inputs.reshape(B * T, D)
    jobs = [(h * H, H, h * H) for h in range(2)]
    for b in range(1, B - 1):
        jobs.append((b * T, T, 0))
    jobs += [((B - 1) * T + h * H, H, h * H) for h in range(2)]
    NJ = len(jobs)

    def body(x_hbm, w_hbm, o_hbm, wvm, xb0, xb1, xb2, xb3, xb4, wsem, isem, osem):
        xbufs = (xb0, xb1, xb2, xb3, xb4)

        def start_in(j):
            r0, n, _ = jobs[j]
            cp = pltpu.make_async_copy(
                x_hbm.at[pl.ds(r0, n)], xbufs[j % _NBUF].at[pl.ds(0, n)],
                isem.at[j % _NBUF])
            cp.start()
            return cp

        def start_out(j):
            r0, n, _ = jobs[j]
            cp = pltpu.make_async_copy(
                xbufs[j % _NBUF].at[pl.ds(0, n)], o_hbm.at[pl.ds(r0, n)],
                osem.at[j % _NBUF])
            cp.start()
            return cp

        wcps = []
        for h in range(2):
            cp = pltpu.make_async_copy(
                w_hbm.at[pl.ds(h * H, H)], wvm.at[pl.ds(h * H, H)],
                wsem.at[h])
            cp.start()
            wcps.append(cp)
        incp = [None] * NJ
        outcp = [None] * NJ
        for j in range(_NBUF - 1):
            incp[j] = start_in(j)
        for j in range(NJ):
            r0, n, woff = jobs[j]
            if j < 2:
                wcps[j].wait()
            incp[j].wait()
            xbuf = xbufs[j % _NBUF]
            xbuf[pl.ds(0, n)] = xbuf[pl.ds(0, n)] + wvm[pl.ds(woff, n)]
            outcp[j] = start_out(j)
            nxt = j + _NBUF - 1
            if nxt < NJ:
                if nxt >= _NBUF:
                    outcp[nxt - _NBUF].wait()
                incp[nxt] = start_in(nxt)
        for j in range(max(0, NJ - _NBUF), NJ):
            outcp[j].wait()

    out2 = pl.pallas_call(
        body,
        grid=(1,),
        in_specs=[
            pl.BlockSpec(memory_space=pl.ANY),
            pl.BlockSpec(memory_space=pl.ANY),
        ],
        out_specs=pl.BlockSpec(memory_space=pl.ANY),
        out_shape=jax.ShapeDtypeStruct((B * T, D), inputs.dtype),
        scratch_shapes=[
            pltpu.VMEM((T, D), jnp.float32),
            pltpu.VMEM((T, D), jnp.float32),
            pltpu.VMEM((T, D), jnp.float32),
            pltpu.VMEM((T, D), jnp.float32),
            pltpu.VMEM((T, D), jnp.float32),
            pltpu.VMEM((T, D), jnp.float32),
            pltpu.SemaphoreType.DMA((2,)),
            pltpu.SemaphoreType.DMA((_NBUF,)),
            pltpu.SemaphoreType.DMA((_NBUF,)),
        ],
    )(x2, embed_weight)
    return out2.reshape(B, T, D)
